# baseline (speedup 1.0000x reference)
import jax
import jax.numpy as jnp
from jax.experimental import pallas as pl
from jax.experimental.pallas import tpu as pltpu

_BM_FACES = 256
_BM_NODES = 128


def _dot(a, b, dims, **kw):
    return jax.lax.dot_general(a, b, (dims, ((), ())),
                               preferred_element_type=jnp.float32, **kw)


def _faces_body(di2_ref, vr_ref, w2cat_ref, b2_ref, f_ref, w1_ref, b1_ref,
                y0_ref, y1_ref, a2_ref):
    m = pl.program_id(1)

    @pl.when(m == 0)
    def _build():
        a2 = _dot(vr_ref[0], w2cat_ref[...], ((1,), (0,)),
                  precision=jax.lax.Precision.HIGHEST)
        a2_ref[...] = a2.astype(jnp.bfloat16)

    di = di2_ref[0].astype(jnp.bfloat16)
    acc = _dot(di[:, 0:2048], a2_ref[:, 0:128], ((1,), (0,)))
    for ci in range(1, 4):
        acc += _dot(di[:, ci * 2048:(ci + 1) * 2048],
                    a2_ref[:, ci * 128:(ci + 1) * 128], ((1,), (0,)))
    y1_ref[0] = acc + b2_ref[...]

    y0 = _dot(f_ref[0].astype(jnp.bfloat16), w1_ref[...].astype(jnp.bfloat16),
              ((1,), (1,)))
    y0_ref[0] = y0 + b1_ref[...]


def _nodes_body(dia2_ref, l_ref, v_ref, fr_ref, w4cat_ref, b4_ref, w3_ref,
                b3_ref, w5_ref, b5_ref, y2_ref, y3_ref, y4_ref,
                a4_ref, c_ref):
    m = pl.program_id(1)

    @pl.when(m == 0)
    def _build():
        a4 = _dot(fr_ref[0], w4cat_ref[...], ((1,), (0,)),
                  precision=jax.lax.Precision.HIGHEST)
        a4_ref[...] = a4.astype(jnp.bfloat16)
        c = _dot(v_ref[0], w5_ref[...], ((1,), (1,)),
                 precision=jax.lax.Precision.HIGHEST)
        c_ref[...] = c.astype(jnp.bfloat16)

    dia = dia2_ref[0].astype(jnp.bfloat16)
    acc = _dot(dia[:, 0:4096], a4_ref[:, 0:128], ((1,), (0,)))
    for ci in range(1, 4):
        acc += _dot(dia[:, ci * 4096:(ci + 1) * 4096],
                    a4_ref[:, ci * 128:(ci + 1) * 128], ((1,), (0,)))
    y3_ref[0] = acc + b4_ref[...]

    vblk = v_ref[0, pl.ds(m * _BM_NODES, _BM_NODES), :]
    y2 = _dot(vblk.astype(jnp.bfloat16), w3_ref[...].astype(jnp.bfloat16),
              ((1,), (1,)))
    y2_ref[0] = y2 + b3_ref[...]

    y4 = _dot(l_ref[0].astype(jnp.bfloat16), c_ref[...], ((1,), (0,)))
    y4_ref[0] = y4 + b5_ref[...]


def _stack_w(w):
    no, ni = w.shape
    return w.reshape(no, 4, ni // 4).transpose(2, 1, 0).reshape(ni // 4, 4 * no)


def kernel(L, Di, DiA, v, f, W1, b1, W2, b2, W3, b3, W4, b4, W5, b5):
    bsz, n_nodes, ni = v.shape
    n_faces = f.shape[1]
    no = W1.shape[0]

    di2 = Di.reshape(bsz, n_faces, 16 * n_nodes)
    dia2 = DiA.reshape(bsz, n_nodes, 16 * n_faces)
    vr = v.reshape(bsz, 4 * n_nodes, ni // 4)
    fr = f.reshape(bsz, 4 * n_faces, ni // 4)
    w2cat = _stack_w(W2)
    w4cat = _stack_w(W4)
    b1r, b2r, b3r, b4r, b5r = (x.reshape(1, no) for x in (b1, b2, b3, b4, b5))

    bm_f, bm_n = _BM_FACES, _BM_NODES
    fixed = lambda b, m: (0, 0)

    y0, y1 = pl.pallas_call(
        _faces_body,
        grid=(bsz, n_faces // bm_f),
        in_specs=[
            pl.BlockSpec((1, bm_f, 16 * n_nodes), lambda b, m: (b, m, 0)),
            pl.BlockSpec((1, 4 * n_nodes, ni // 4), lambda b, m: (b, 0, 0)),
            pl.BlockSpec((ni // 4, 4 * no), fixed),
            pl.BlockSpec((1, no), fixed),
            pl.BlockSpec((1, bm_f, ni), lambda b, m: (b, m, 0)),
            pl.BlockSpec((no, ni), fixed),
            pl.BlockSpec((1, no), fixed),
        ],
        out_specs=[
            pl.BlockSpec((1, bm_f, no), lambda b, m: (b, m, 0)),
            pl.BlockSpec((1, bm_f, no), lambda b, m: (b, m, 0)),
        ],
        out_shape=[
            jax.ShapeDtypeStruct((bsz, n_faces, no), jnp.float32),
            jax.ShapeDtypeStruct((bsz, n_faces, no), jnp.float32),
        ],
        scratch_shapes=[pltpu.VMEM((4 * n_nodes, 4 * no), jnp.bfloat16)],
    )(di2, vr, w2cat, b2r, f, W1, b1r)

    y2, y3, y4 = pl.pallas_call(
        _nodes_body,
        grid=(bsz, n_nodes // bm_n),
        in_specs=[
            pl.BlockSpec((1, bm_n, 16 * n_faces), lambda b, m: (b, m, 0)),
            pl.BlockSpec((1, bm_n, n_nodes), lambda b, m: (b, m, 0)),
            pl.BlockSpec((1, n_nodes, ni), lambda b, m: (b, 0, 0)),
            pl.BlockSpec((1, 4 * n_faces, ni // 4), lambda b, m: (b, 0, 0)),
            pl.BlockSpec((ni // 4, 4 * no), fixed),
            pl.BlockSpec((1, no), fixed),
            pl.BlockSpec((no, ni), fixed),
            pl.BlockSpec((1, no), fixed),
            pl.BlockSpec((no, ni), fixed),
            pl.BlockSpec((1, no), fixed),
        ],
        out_specs=[
            pl.BlockSpec((1, bm_n, no), lambda b, m: (b, m, 0)),
            pl.BlockSpec((1, bm_n, no), lambda b, m: (b, m, 0)),
            pl.BlockSpec((1, bm_n, no), lambda b, m: (b, m, 0)),
        ],
        out_shape=[
            jax.ShapeDtypeStruct((bsz, n_nodes, no), jnp.float32),
            jax.ShapeDtypeStruct((bsz, n_nodes, no), jnp.float32),
            jax.ShapeDtypeStruct((bsz, n_nodes, no), jnp.float32),
        ],
        scratch_shapes=[
            pltpu.VMEM((4 * n_faces, 4 * no), jnp.bfloat16),
            pltpu.VMEM((n_nodes, no), jnp.bfloat16),
        ],
    )(dia2, L, v, fr, w4cat, b4r, W3, b3r, W5, b5r)

    return (y0, y1, y2, y3, y4)


# speedup vs baseline: 3.2354x; 3.2354x over previous
import jax
import jax.numpy as jnp
from jax.experimental import pallas as pl
from jax.experimental.pallas import tpu as pltpu

_BF = jnp.bfloat16


def _dot(a, b, dims=((1,), (0,))):
    return jax.lax.dot_general(a.astype(_BF), b.astype(_BF),
                               (dims, ((), ())),
                               preferred_element_type=jnp.float32)


def _f_body(di_ref, vr_ref, t1_ref):
    t1_ref[0] = _dot(di_ref[0], vr_ref[0])


def _n_body(dia_ref, fr_ref, l_ref, v_ref, w5_ref, b5_ref,
            t3_ref, y4_ref, c_ref):
    m = pl.program_id(1)

    @pl.when(m == 0)
    def _build():
        c_ref[...] = _dot(v_ref[0], w5_ref[...], ((1,), (1,))).astype(_BF)

    t3_ref[0] = _dot(dia_ref[0], fr_ref[0])
    y4_ref[0] = _dot(l_ref[0], c_ref[...]) + b5_ref[...]


def _s_body(t1_ref, t3_ref, f_ref, v_ref, w1_ref, b1_ref, w2_ref, b2_ref,
            w3_ref, b3_ref, w4_ref, b4_ref, y0_ref, y1_ref, y2_ref, y3_ref):
    y0_ref[0] = _dot(f_ref[0], w1_ref[...], ((1,), (1,))) + b1_ref[...]
    y1_ref[0] = _dot(t1_ref[0], w2_ref[...], ((1,), (1,))) + b2_ref[...]
    y2_ref[0] = _dot(v_ref[0], w3_ref[...], ((1,), (1,))) + b3_ref[...]
    y3_ref[0] = _dot(t3_ref[0], w4_ref[...], ((1,), (1,))) + b4_ref[...]


def kernel(L, Di, DiA, v, f, W1, b1, W2, b2, W3, b3, W4, b4, W5, b5):
    bsz, n_nodes, ni = v.shape
    n_faces = f.shape[1]
    no = W1.shape[0]
    nc = ni // 4

    vr = v.reshape(bsz, 4 * n_nodes, nc)
    fr = f.reshape(bsz, 4 * n_faces, nc)
    b1r, b2r, b3r, b4r, b5r = (x.reshape(1, no) for x in (b1, b2, b3, b4, b5))
    fixed = lambda b, m: (0, 0)

    bm = 512
    t1 = pl.pallas_call(
        _f_body,
        grid=(bsz, 4 * n_faces // bm),
        in_specs=[
            pl.BlockSpec((1, bm, 4 * n_nodes), lambda b, m: (b, m, 0)),
            pl.BlockSpec((1, 4 * n_nodes, nc), lambda b, m: (b, 0, 0)),
        ],
        out_specs=pl.BlockSpec((1, bm, nc), lambda b, m: (b, m, 0)),
        out_shape=jax.ShapeDtypeStruct((bsz, 4 * n_faces, nc), jnp.float32),
    )(Di, vr)

    bl = n_nodes // 4
    t3, y4 = pl.pallas_call(
        _n_body,
        grid=(bsz, 4 * n_nodes // bm),
        in_specs=[
            pl.BlockSpec((1, bm, 4 * n_faces), lambda b, m: (b, m, 0)),
            pl.BlockSpec((1, 4 * n_faces, nc), lambda b, m: (b, 0, 0)),
            pl.BlockSpec((1, bl, n_nodes), lambda b, m: (b, m, 0)),
            pl.BlockSpec((1, n_nodes, ni), lambda b, m: (b, 0, 0)),
            pl.BlockSpec((no, ni), fixed),
            pl.BlockSpec((1, no), fixed),
        ],
        out_specs=[
            pl.BlockSpec((1, bm, nc), lambda b, m: (b, m, 0)),
            pl.BlockSpec((1, bl, no), lambda b, m: (b, m, 0)),
        ],
        out_shape=[
            jax.ShapeDtypeStruct((bsz, 4 * n_nodes, nc), jnp.float32),
            jax.ShapeDtypeStruct((bsz, n_nodes, no), jnp.float32),
        ],
        scratch_shapes=[pltpu.VMEM((n_nodes, no), _BF)],
    )(DiA, fr, L, v, W5, b5r)

    t1r = t1.reshape(bsz, n_faces, ni)
    t3r = t3.reshape(bsz, n_nodes, ni)

    y0, y1, y2, y3 = pl.pallas_call(
        _s_body,
        grid=(bsz,),
        in_specs=[
            pl.BlockSpec((1, n_faces, ni), lambda b: (b, 0, 0)),
            pl.BlockSpec((1, n_nodes, ni), lambda b: (b, 0, 0)),
            pl.BlockSpec((1, n_faces, ni), lambda b: (b, 0, 0)),
            pl.BlockSpec((1, n_nodes, ni), lambda b: (b, 0, 0)),
            pl.BlockSpec((no, ni), lambda b: (0, 0)),
            pl.BlockSpec((1, no), lambda b: (0, 0)),
            pl.BlockSpec((no, ni), lambda b: (0, 0)),
            pl.BlockSpec((1, no), lambda b: (0, 0)),
            pl.BlockSpec((no, ni), lambda b: (0, 0)),
            pl.BlockSpec((1, no), lambda b: (0, 0)),
            pl.BlockSpec((no, ni), lambda b: (0, 0)),
            pl.BlockSpec((1, no), lambda b: (0, 0)),
        ],
        out_specs=[
            pl.BlockSpec((1, n_faces, no), lambda b: (b, 0, 0)),
            pl.BlockSpec((1, n_faces, no), lambda b: (b, 0, 0)),
            pl.BlockSpec((1, n_nodes, no), lambda b: (b, 0, 0)),
            pl.BlockSpec((1, n_nodes, no), lambda b: (b, 0, 0)),
        ],
        out_shape=[
            jax.ShapeDtypeStruct((bsz, n_faces, no), jnp.float32),
            jax.ShapeDtypeStruct((bsz, n_faces, no), jnp.float32),
            jax.ShapeDtypeStruct((bsz, n_nodes, no), jnp.float32),
            jax.ShapeDtypeStruct((bsz, n_nodes, no), jnp.float32),
        ],
    )(t1r, t3r, f, v, W1, b1r, W2, b2r, W3, b3r, W4, b4r)

    return (y0, y1, y2, y3, y4)
